# Initial kernel scaffold; baseline (speedup 1.0000x reference)
import jax
import jax.numpy as jnp
from jax.experimental import pallas as pl


def kernel(x, edge_index, W1l, b1l, W1r, W2l, b2l, W2r, Wlin, blin):
    raise NotImplementedError("write your pallas kernel here")


# speedup vs baseline: 7.8748x; 7.8748x over previous
import functools
import jax
import jax.numpy as jnp
from jax import lax
from jax.experimental import pallas as pl
from jax.experimental.pallas import tpu as pltpu
from jax.experimental.pallas import tpu_sc as plsc

N = 10000
E = 320000
D = 128
NC = 2
NS = 16
NW = NC * NS
EPW = E // NW
CHUNK = 80
NCHUNK = EPW // CHUNK
STRIPE = 624
TAIL = N - STRIPE * NS
TAIL_OFF = STRIPE * NS

_MESH = plsc.VectorSubcoreMesh(core_axis_name="c", subcore_axis_name="s")


def _sc_agg_body(with_deg, *refs):
    if with_deg:
        (table, src, dst, zrows, zdeg, out, deg_out,
         idx_s, idx_d, rows, ones_v, acc, deg_sh, sem) = refs
    else:
        (table, src, dst, zrows, out,
         idx_s, idx_d, rows, acc, sem) = refs
    c = lax.axis_index("c")
    s = lax.axis_index("s")
    wid = s * NC + c

    pltpu.sync_copy(zrows.at[pl.ds(s * STRIPE, STRIPE)],
                    acc.at[pl.ds(s * STRIPE, STRIPE)])

    @pl.when(s == 0)
    def _():
        pltpu.sync_copy(zrows.at[pl.ds(TAIL_OFF, TAIL)],
                        acc.at[pl.ds(TAIL_OFF, TAIL)])
    if with_deg:
        @pl.when(s == 0)
        def _():
            pltpu.sync_copy(zdeg, deg_sh)
        for i in range(CHUNK // 16):
            ones_v[pl.ds(i * 16, 16)] = jnp.ones((16,), jnp.float32)

    pltpu.sync_copy(src.at[wid], idx_s)
    pltpu.sync_copy(dst.at[wid], idx_d)
    plsc.subcore_barrier()

    @pl.loop(0, NCHUNK)
    def _(j):
        pltpu.async_copy(table.at[idx_s.at[j]], rows, sem).wait()
        pltpu.sync_copy(rows, acc.at[idx_d.at[j]], add=True)
        if with_deg:
            pltpu.sync_copy(ones_v, deg_sh.at[idx_d.at[j]], add=True)

    plsc.subcore_barrier()

    pltpu.sync_copy(acc.at[pl.ds(s * STRIPE, STRIPE)],
                    out.at[c, pl.ds(s * STRIPE, STRIPE)])

    @pl.when(s == 0)
    def _():
        pltpu.sync_copy(acc.at[pl.ds(TAIL_OFF, TAIL)],
                        out.at[c, pl.ds(TAIL_OFF, TAIL)])
    if with_deg:
        @pl.when(s == 0)
        def _():
            pltpu.sync_copy(deg_sh, deg_out.at[c])


def _make_sc_agg(with_deg):
    out_type = [jax.ShapeDtypeStruct((NC, N, D), jnp.float32)]
    if with_deg:
        out_type.append(jax.ShapeDtypeStruct((NC, N), jnp.float32))
    scratch = [
        pltpu.VMEM((NCHUNK, CHUNK), jnp.int32),
        pltpu.VMEM((NCHUNK, CHUNK), jnp.int32),
        pltpu.VMEM((CHUNK, D), jnp.float32),
    ]
    if with_deg:
        scratch.append(pltpu.VMEM((CHUNK,), jnp.float32))
    scratch.append(pltpu.VMEM_SHARED((N, D), jnp.float32))
    if with_deg:
        scratch.append(pltpu.VMEM_SHARED((N,), jnp.float32))
    scratch.append(pltpu.SemaphoreType.DMA)
    return pl.kernel(
        functools.partial(_sc_agg_body, with_deg),
        out_type=tuple(out_type),
        mesh=_MESH,
        scratch_types=scratch,
    )


_sc_agg_l1 = _make_sc_agg(True)
_sc_agg_l2 = _make_sc_agg(False)


def _dense_body(sum_ref, deg_ref, x_ref, wl_ref, bl_ref, wr_ref, o_ref):
    ssum = sum_ref[0] + sum_ref[1]
    deg = deg_ref[0] + deg_ref[1]
    recip = 1.0 / jnp.maximum(deg, 1.0)
    mean = ssum * recip[:, None]
    acc = jnp.dot(mean, wl_ref[...], preferred_element_type=jnp.float32)
    acc += jnp.dot(x_ref[...], wr_ref[...], preferred_element_type=jnp.float32)
    o_ref[...] = jnp.maximum(acc + bl_ref[...][None, :], 0.0)


def _final_body(sum_ref, deg_ref, h_ref, wl_ref, bl_ref, wr_ref,
                wlin_ref, blin_ref, o_ref):
    ssum = sum_ref[0] + sum_ref[1]
    deg = deg_ref[0] + deg_ref[1]
    recip = 1.0 / jnp.maximum(deg, 1.0)
    mean = ssum * recip[:, None]
    acc = jnp.dot(mean, wl_ref[...], preferred_element_type=jnp.float32)
    acc += jnp.dot(h_ref[...], wr_ref[...], preferred_element_type=jnp.float32)
    h2 = jnp.maximum(acc + bl_ref[...][None, :], 0.0)
    o = jnp.dot(h2, wlin_ref[...], preferred_element_type=jnp.float32)
    o = o + blin_ref[...][None, :]
    m = jnp.max(o, axis=1, keepdims=True)
    lse = jnp.log(jnp.sum(jnp.exp(o - m), axis=1, keepdims=True)) + m
    o_ref[...] = o - lse


_tc_dense = pl.pallas_call(
    _dense_body,
    out_shape=jax.ShapeDtypeStruct((N, D), jnp.float32),
)

_tc_final = pl.pallas_call(
    _final_body,
    out_shape=jax.ShapeDtypeStruct((N, 2), jnp.float32),
)


def kernel(x, edge_index, W1l, b1l, W1r, W2l, b2l, W2r, Wlin, blin):
    src = edge_index[0].reshape(NW, NCHUNK, CHUNK)
    dst = edge_index[1].reshape(NW, NCHUNK, CHUNK)
    zrows = jnp.zeros((N, D), jnp.float32)
    zdeg = jnp.zeros((N,), jnp.float32)

    sum1, deg = _sc_agg_l1(x, src, dst, zrows, zdeg)
    h = _tc_dense(sum1, deg, x, W1l, b1l, W1r)
    (sum2,) = _sc_agg_l2(h, src, dst, zrows)
    return _tc_final(sum2, deg, h, W2l, b2l, W2r, Wlin, blin)


# speedup vs baseline: 11.8289x; 1.5021x over previous
import functools
import jax
import jax.numpy as jnp
from jax import lax
from jax.experimental import pallas as pl
from jax.experimental.pallas import tpu as pltpu
from jax.experimental.pallas import tpu_sc as plsc

N = 10000
E = 320000
D = 128
NC = 2
NS = 16
NW = NC * NS
EPW = E // NW
CHUNK = 80
NCHUNK = EPW // CHUNK
GCH = 25
NGROUP = NCHUNK // GCH
STRIPE = 624
TAIL = N - STRIPE * NS
TAIL_OFF = STRIPE * NS

_MESH = plsc.VectorSubcoreMesh(core_axis_name="c", subcore_axis_name="s")


def _sc_agg_body(with_deg, *refs):
    if with_deg:
        (table, src, dst, zrows, zdeg, out, deg_out,
         idx_s, idx_d, rows, ones_v, acc, deg_sh, sem0, sem1) = refs
    else:
        (table, src, dst, zrows, out,
         idx_s, idx_d, rows, acc, sem0, sem1) = refs
    c = lax.axis_index("c")
    s = lax.axis_index("s")
    wid = s * NC + c

    pltpu.sync_copy(zrows.at[pl.ds(s * STRIPE, STRIPE)],
                    acc.at[pl.ds(s * STRIPE, STRIPE)])

    @pl.when(s == 0)
    def _():
        pltpu.sync_copy(zrows.at[pl.ds(TAIL_OFF, TAIL)],
                        acc.at[pl.ds(TAIL_OFF, TAIL)])
    if with_deg:
        @pl.when(s == 0)
        def _():
            pltpu.sync_copy(zdeg, deg_sh)
        for i in range(CHUNK // 16):
            ones_v[pl.ds(i * 16, 16)] = jnp.ones((16,), jnp.float32)

    plsc.subcore_barrier()

    @pl.loop(0, NGROUP)
    def _(g):
        pltpu.sync_copy(src.at[wid, g], idx_s)
        pltpu.sync_copy(dst.at[wid, g], idx_d)
        pltpu.async_copy(table.at[idx_s.at[0]], rows.at[0], sem0)

        @pl.loop(0, GCH, step=2)
        def _(j):
            @pl.when(j + 1 < GCH)
            def _():
                pltpu.async_copy(table.at[idx_s.at[j + 1]], rows.at[1], sem1)

            pltpu.make_async_copy(
                table.at[idx_s.at[j]], rows.at[0], sem0).wait()
            pltpu.sync_copy(rows.at[0], acc.at[idx_d.at[j]], add=True)
            if with_deg:
                pltpu.sync_copy(ones_v, deg_sh.at[idx_d.at[j]], add=True)

            @pl.when(j + 2 < GCH)
            def _():
                pltpu.async_copy(table.at[idx_s.at[j + 2]], rows.at[0], sem0)

            @pl.when(j + 1 < GCH)
            def _():
                pltpu.make_async_copy(
                    table.at[idx_s.at[j + 1]], rows.at[1], sem1).wait()
                pltpu.sync_copy(rows.at[1], acc.at[idx_d.at[j + 1]], add=True)
                if with_deg:
                    pltpu.sync_copy(ones_v,
                                    deg_sh.at[idx_d.at[j + 1]], add=True)

    plsc.subcore_barrier()

    pltpu.sync_copy(acc.at[pl.ds(s * STRIPE, STRIPE)],
                    out.at[c, pl.ds(s * STRIPE, STRIPE)])

    @pl.when(s == 0)
    def _():
        pltpu.sync_copy(acc.at[pl.ds(TAIL_OFF, TAIL)],
                        out.at[c, pl.ds(TAIL_OFF, TAIL)])
    if with_deg:
        @pl.when(s == 0)
        def _():
            pltpu.sync_copy(deg_sh, deg_out.at[c])


def _make_sc_agg(with_deg):
    out_type = [jax.ShapeDtypeStruct((NC, N, D), jnp.float32)]
    if with_deg:
        out_type.append(jax.ShapeDtypeStruct((NC, N), jnp.float32))
    scratch = [
        pltpu.VMEM((GCH, CHUNK), jnp.int32),
        pltpu.VMEM((GCH, CHUNK), jnp.int32),
        pltpu.VMEM((2, CHUNK, D), jnp.float32),
    ]
    if with_deg:
        scratch.append(pltpu.VMEM((CHUNK,), jnp.float32))
    scratch.append(pltpu.VMEM_SHARED((N, D), jnp.float32))
    if with_deg:
        scratch.append(pltpu.VMEM_SHARED((N,), jnp.float32))
    scratch.append(pltpu.SemaphoreType.DMA)
    scratch.append(pltpu.SemaphoreType.DMA)
    return pl.kernel(
        functools.partial(_sc_agg_body, with_deg),
        out_type=tuple(out_type),
        mesh=_MESH,
        scratch_types=scratch,
    )


_sc_agg_l1 = _make_sc_agg(True)
_sc_agg_l2 = _make_sc_agg(False)


def _dense_body(sum_ref, deg_ref, x_ref, wl_ref, bl_ref, wr_ref, o_ref):
    ssum = sum_ref[0] + sum_ref[1]
    deg = deg_ref[0] + deg_ref[1]
    recip = 1.0 / jnp.maximum(deg, 1.0)
    mean = ssum * recip[:, None]
    acc = jnp.dot(mean, wl_ref[...], preferred_element_type=jnp.float32)
    acc += jnp.dot(x_ref[...], wr_ref[...], preferred_element_type=jnp.float32)
    o_ref[...] = jnp.maximum(acc + bl_ref[...][None, :], 0.0)


def _final_body(sum_ref, deg_ref, h_ref, wl_ref, bl_ref, wr_ref,
                wlin_ref, blin_ref, o_ref):
    ssum = sum_ref[0] + sum_ref[1]
    deg = deg_ref[0] + deg_ref[1]
    recip = 1.0 / jnp.maximum(deg, 1.0)
    mean = ssum * recip[:, None]
    acc = jnp.dot(mean, wl_ref[...], preferred_element_type=jnp.float32)
    acc += jnp.dot(h_ref[...], wr_ref[...], preferred_element_type=jnp.float32)
    h2 = jnp.maximum(acc + bl_ref[...][None, :], 0.0)
    o = jnp.dot(h2, wlin_ref[...], preferred_element_type=jnp.float32)
    o = o + blin_ref[...][None, :]
    m = jnp.max(o, axis=1, keepdims=True)
    lse = jnp.log(jnp.sum(jnp.exp(o - m), axis=1, keepdims=True)) + m
    o_ref[...] = o - lse


_tc_dense = pl.pallas_call(
    _dense_body,
    out_shape=jax.ShapeDtypeStruct((N, D), jnp.float32),
)

_tc_final = pl.pallas_call(
    _final_body,
    out_shape=jax.ShapeDtypeStruct((N, 2), jnp.float32),
)


def kernel(x, edge_index, W1l, b1l, W1r, W2l, b2l, W2r, Wlin, blin):
    src = edge_index[0].reshape(NW, NGROUP, GCH, CHUNK)
    dst = edge_index[1].reshape(NW, NGROUP, GCH, CHUNK)
    zrows = jnp.zeros((N, D), jnp.float32)
    zdeg = jnp.zeros((N,), jnp.float32)

    sum1, deg = _sc_agg_l1(x, src, dst, zrows, zdeg)
    h = _tc_dense(sum1, deg, x, W1l, b1l, W1r)
    (sum2,) = _sc_agg_l2(h, src, dst, zrows)
    return _tc_final(sum2, deg, h, W2l, b2l, W2r, Wlin, blin)


# speedup vs baseline: 13.4098x; 1.1337x over previous
import functools
import jax
import jax.numpy as jnp
from jax import lax
from jax.experimental import pallas as pl
from jax.experimental.pallas import tpu as pltpu
from jax.experimental.pallas import tpu_sc as plsc

N = 10000
E = 320000
D = 128
NC = 2
NS = 16
NW = NC * NS
CHUNK = 80
SLAB = 8
NSLAB = E // (SLAB * CHUNK)
SL_BASE = NSLAB // NW
SL_EXTRA = NSLAB % NW
STRIPE = 624
TAIL = N - STRIPE * NS
TAIL_OFF = STRIPE * NS

_MESH = plsc.VectorSubcoreMesh(core_axis_name="c", subcore_axis_name="s")


def _sc_agg_body(with_deg, *refs):
    if with_deg:
        (table, src, dst, zrows, zdeg, out, deg_out,
         idx_s, idx_d, rows, ones_v, acc, deg_sh, semg, sems, semi) = refs
    else:
        (table, src, dst, zrows, out,
         idx_s, idx_d, rows, acc, semg, sems, semi) = refs
    c = lax.axis_index("c")
    s = lax.axis_index("s")
    wid = s * NC + c

    pltpu.sync_copy(zrows.at[pl.ds(s * STRIPE, STRIPE)],
                    acc.at[pl.ds(s * STRIPE, STRIPE)])

    @pl.when(s == 0)
    def _():
        pltpu.sync_copy(zrows.at[pl.ds(TAIL_OFF, TAIL)],
                        acc.at[pl.ds(TAIL_OFF, TAIL)])
    if with_deg:
        @pl.when(s == 0)
        def _():
            pltpu.sync_copy(zdeg, deg_sh)
        for i in range(CHUNK // 16):
            ones_v[pl.ds(i * 16, 16)] = jnp.ones((16,), jnp.float32)

    plsc.subcore_barrier()

    nsl = SL_BASE + jnp.where(wid < SL_EXTRA, 1, 0)
    sl0 = SL_BASE * wid + jnp.minimum(wid, SL_EXTRA)
    nchunks = nsl * SLAB

    def slab_ld(m):
        slot = lax.rem(m, 2)
        pltpu.async_copy(src.at[sl0 + m], idx_s.at[slot], semi)
        pltpu.async_copy(dst.at[sl0 + m], idx_d.at[slot], semi)

    def slab_wt(m):
        slot = lax.rem(m, 2)
        pltpu.make_async_copy(src.at[sl0 + m], idx_s.at[slot], semi).wait()
        pltpu.make_async_copy(dst.at[sl0 + m], idx_d.at[slot], semi).wait()

    def i_ref(arr, q):
        return arr.at[lax.rem(lax.div(q, SLAB), 2), lax.rem(q, SLAB)]

    def g_start(q, b):
        pltpu.async_copy(table.at[i_ref(idx_s, q)], rows.at[b], semg.at[b])

    def g_wait(q, b):
        pltpu.make_async_copy(
            table.at[i_ref(idx_s, q)], rows.at[b], semg.at[b]).wait()

    def s_start(q, b):
        pltpu.async_copy(rows.at[b], acc.at[i_ref(idx_d, q)],
                         sems.at[b], add=True)
        if with_deg:
            pltpu.sync_copy(ones_v, deg_sh.at[i_ref(idx_d, q)], add=True)

    def s_wait(q, b):
        pltpu.make_async_copy(rows.at[b], acc.at[i_ref(idx_d, q)],
                              sems.at[b]).wait()

    slab_ld(0)
    slab_wt(0)
    g_start(0, 0)
    g_start(1, 1)

    @pl.loop(0, nchunks, step=4)
    def _(q0):
        for kk in range(4):
            q = q0 + kk
            b = kk

            g_wait(q, b)
            s_start(q, b)

            @pl.when(q >= 2)
            def _():
                s_wait(q - 2, (b + 2) % 4)

            @pl.when(lax.rem(q, SLAB) == 2)
            def _():
                m = lax.div(q, SLAB) + 1

                @pl.when(m < nsl)
                def _():
                    slab_ld(m)

            @pl.when(q + 2 < nchunks)
            def _():
                @pl.when(lax.rem(q + 2, SLAB) == 0)
                def _():
                    slab_wt(lax.div(q + 2, SLAB))
                g_start(q + 2, (b + 2) % 4)

    s_wait(nchunks - 2, 2)
    s_wait(nchunks - 1, 3)

    plsc.subcore_barrier()

    pltpu.sync_copy(acc.at[pl.ds(s * STRIPE, STRIPE)],
                    out.at[c, pl.ds(s * STRIPE, STRIPE)])

    @pl.when(s == 0)
    def _():
        pltpu.sync_copy(acc.at[pl.ds(TAIL_OFF, TAIL)],
                        out.at[c, pl.ds(TAIL_OFF, TAIL)])
    if with_deg:
        @pl.when(s == 0)
        def _():
            pltpu.sync_copy(deg_sh, deg_out.at[c])


def _make_sc_agg(with_deg):
    out_type = [jax.ShapeDtypeStruct((NC, N, D), jnp.float32)]
    if with_deg:
        out_type.append(jax.ShapeDtypeStruct((NC, N), jnp.float32))
    scratch = [
        pltpu.VMEM((2, SLAB, CHUNK), jnp.int32),
        pltpu.VMEM((2, SLAB, CHUNK), jnp.int32),
        pltpu.VMEM((4, CHUNK, D), jnp.float32),
    ]
    if with_deg:
        scratch.append(pltpu.VMEM((CHUNK,), jnp.float32))
    scratch.append(pltpu.VMEM_SHARED((N, D), jnp.float32))
    if with_deg:
        scratch.append(pltpu.VMEM_SHARED((N,), jnp.float32))
    scratch.append(pltpu.SemaphoreType.DMA((4,)))
    scratch.append(pltpu.SemaphoreType.DMA((4,)))
    scratch.append(pltpu.SemaphoreType.DMA)
    return pl.kernel(
        functools.partial(_sc_agg_body, with_deg),
        out_type=tuple(out_type),
        mesh=_MESH,
        scratch_types=scratch,
    )


_sc_agg_l1 = _make_sc_agg(True)
_sc_agg_l2 = _make_sc_agg(False)


def _dense_body(sum_ref, deg_ref, x_ref, wl_ref, bl_ref, wr_ref, o_ref):
    ssum = sum_ref[0] + sum_ref[1]
    deg = deg_ref[0] + deg_ref[1]
    recip = 1.0 / jnp.maximum(deg, 1.0)
    mean = ssum * recip[:, None]
    acc = jnp.dot(mean, wl_ref[...], preferred_element_type=jnp.float32)
    acc += jnp.dot(x_ref[...], wr_ref[...], preferred_element_type=jnp.float32)
    o_ref[...] = jnp.maximum(acc + bl_ref[...][None, :], 0.0)


def _final_body(sum_ref, deg_ref, h_ref, wl_ref, bl_ref, wr_ref,
                wlin_ref, blin_ref, o_ref):
    ssum = sum_ref[0] + sum_ref[1]
    deg = deg_ref[0] + deg_ref[1]
    recip = 1.0 / jnp.maximum(deg, 1.0)
    mean = ssum * recip[:, None]
    acc = jnp.dot(mean, wl_ref[...], preferred_element_type=jnp.float32)
    acc += jnp.dot(h_ref[...], wr_ref[...], preferred_element_type=jnp.float32)
    h2 = jnp.maximum(acc + bl_ref[...][None, :], 0.0)
    o = jnp.dot(h2, wlin_ref[...], preferred_element_type=jnp.float32)
    o = o + blin_ref[...][None, :]
    m = jnp.max(o, axis=1, keepdims=True)
    lse = jnp.log(jnp.sum(jnp.exp(o - m), axis=1, keepdims=True)) + m
    o_ref[...] = o - lse


_tc_dense = pl.pallas_call(
    _dense_body,
    out_shape=jax.ShapeDtypeStruct((N, D), jnp.float32),
)

_tc_final = pl.pallas_call(
    _final_body,
    out_shape=jax.ShapeDtypeStruct((N, 2), jnp.float32),
)


def kernel(x, edge_index, W1l, b1l, W1r, W2l, b2l, W2r, Wlin, blin):
    src = edge_index[0].reshape(NSLAB, SLAB, CHUNK)
    dst = edge_index[1].reshape(NSLAB, SLAB, CHUNK)
    zrows = jnp.zeros((N, D), jnp.float32)
    zdeg = jnp.zeros((N,), jnp.float32)

    sum1, deg = _sc_agg_l1(x, src, dst, zrows, zdeg)
    h = _tc_dense(sum1, deg, x, W1l, b1l, W1r)
    (sum2,) = _sc_agg_l2(h, src, dst, zrows)
    return _tc_final(sum2, deg, h, W2l, b2l, W2r, Wlin, blin)
